# baseline (speedup 1.0000x reference)
import functools
import math

import jax
import jax.numpy as jnp
from jax import lax
from jax.experimental import pallas as pl
from jax.experimental.pallas import tpu as pltpu
from jax.experimental.pallas import tpu_sc as plsc

_VOCAB = 100000
_D = 2048
_SL = 16
_NC = 2
_NS = 16
_NW = _NC * _NS
_B = 8192
_BPW = _B // _NW
_CH = 16
_NCHUNK = _BPW // _CH
_SCALE = math.sqrt(_D)

_mesh = plsc.VectorSubcoreMesh(core_axis_name="c", subcore_axis_name="s")


@functools.partial(
    pl.kernel,
    mesh=_mesh,
    out_type=jax.ShapeDtypeStruct((_B // 8, 8, _D), jnp.float32),
    scratch_types=[
        pltpu.VMEM((_BPW,), jnp.int32),
        pltpu.VMEM((_CH, _SL, 128), jnp.bfloat16),
        pltpu.VMEM((_CH, _SL, 128), jnp.bfloat16),
        pltpu.VMEM((_CH // 8, 8, _D), jnp.float32),
        pltpu.VMEM((_CH // 8, 8, _D), jnp.float32),
        pltpu.SemaphoreType.DMA,
        pltpu.SemaphoreType.DMA,
        pltpu.SemaphoreType.DMA,
        pltpu.SemaphoreType.DMA,
    ],
)
def _embed_sc(idx_hbm, table_hbm, out_hbm, idx_v, gbuf0, gbuf1,
              fbuf0, fbuf1, gsem0, gsem1, osem0, osem1):
    wid = lax.axis_index("s") * _NC + lax.axis_index("c")

    gbufs = (gbuf0, gbuf1)
    fbufs = (fbuf0, fbuf1)
    gsems = (gsem0, gsem1)
    osems = (osem0, osem1)

    pltpu.sync_copy(idx_hbm.at[wid], idx_v)

    scale = jnp.float32(_SCALE)

    def start_gather(i):
        b = i % 2
        v = idx_v[pl.ds(i * _CH, 16)]
        return [pltpu.async_copy(table_hbm.at[v[k]], gbufs[b].at[k], gsems[b])
                for k in range(_CH)]

    def convert_chunk(b):
        gbuf, fbuf = gbufs[b], fbufs[b]

        def row_body(r, _):
            maj = r >> 3
            sub = r & 7
            for s in range(0, _SL, 2):
                for c in range(8):
                    z2 = gbuf[r, pl.ds(s, 2), pl.ds(c * 16, 16)]
                    f2 = z2.astype(jnp.float32)
                    fbuf[maj, sub, pl.ds(s * 128 + c * 16, 16)] = (
                        f2[0] * scale)
                    fbuf[maj, sub, pl.ds((s + 1) * 128 + c * 16, 16)] = (
                        f2[1] * scale)
            return 0
        lax.fori_loop(0, _CH, row_body, 0)

    gh = [None, None]
    oh = [None, None]
    gh[0] = start_gather(0)

    for i in range(_NCHUNK):
        b = i % 2
        nb = (i + 1) % 2
        if i + 1 < _NCHUNK:
            if oh[nb] is not None:
                oh[nb].wait()
            gh[nb] = start_gather(i + 1)
        for h in gh[b]:
            h.wait()
        convert_chunk(b)
        oh[b] = pltpu.async_copy(
            fbufs[b], out_hbm.at[pl.ds(wid * (_BPW // 8) + i * 2, 2)],
            osems[b])

    oh[0].wait()
    oh[1].wait()


def kernel(input_ids, embed_table):
    idx = input_ids.reshape(_NW, _BPW)
    table = embed_table.reshape(_VOCAB, _SL, 128)
    out = _embed_sc(idx, table)
    batch, seq = input_ids.shape
    return out.reshape(batch, seq, _D)


# speedup vs baseline: 1.1852x; 1.1852x over previous
import functools
import math

import jax
import jax.numpy as jnp
from jax import lax
from jax.experimental import pallas as pl
from jax.experimental.pallas import tpu as pltpu
from jax.experimental.pallas import tpu_sc as plsc

_VOCAB = 100000
_D = 2048
_SL = 16
_NC = 2
_NS = 16
_NW = _NC * _NS
_B = 8192
_BPW = _B // _NW
_CH = 32
_NCHUNK = _BPW // _CH
_SCALE = math.sqrt(_D)

_mesh = plsc.VectorSubcoreMesh(core_axis_name="c", subcore_axis_name="s")


@functools.partial(
    pl.kernel,
    mesh=_mesh,
    out_type=jax.ShapeDtypeStruct((_B, _SL, 128), jnp.bfloat16),
    scratch_types=[
        pltpu.VMEM((_BPW,), jnp.int32),
        pltpu.VMEM((_CH, _SL, 128), jnp.bfloat16),
        pltpu.VMEM((_CH, _SL, 128), jnp.bfloat16),
        pltpu.SemaphoreType.DMA,
        pltpu.SemaphoreType.DMA,
        pltpu.SemaphoreType.DMA,
        pltpu.SemaphoreType.DMA,
    ],
)
def _embed_sc(idx_hbm, table_hbm, out_hbm, idx_v, buf0, buf1,
              gsem0, gsem1, osem0, osem1):
    wid = lax.axis_index("s") * _NC + lax.axis_index("c")
    base = wid * _BPW

    bufs = (buf0, buf1)
    gsems = (gsem0, gsem1)
    osems = (osem0, osem1)

    pltpu.sync_copy(idx_hbm.at[wid], idx_v)

    def start_gather(i):
        b = i % 2
        handles = []
        for g in range(_CH // 16):
            v = idx_v[pl.ds(i * _CH + g * 16, 16)]
            for k in range(16):
                handles.append(pltpu.async_copy(
                    table_hbm.at[v[k]], bufs[b].at[g * 16 + k], gsems[b]))
        return handles

    gh = [None, None]
    oh = [None, None]
    gh[0] = start_gather(0)

    for i in range(_NCHUNK):
        b = i % 2
        nb = (i + 1) % 2
        if i + 1 < _NCHUNK:
            if oh[nb] is not None:
                oh[nb].wait()
            gh[nb] = start_gather(i + 1)
        for h in gh[b]:
            h.wait()
        oh[b] = pltpu.async_copy(
            bufs[b], out_hbm.at[pl.ds(base + i * _CH, _CH)], osems[b])

    oh[0].wait()
    oh[1].wait()


def _scale_body(x_ref, o_ref):
    x = x_ref[...] * jnp.bfloat16(_SCALE)
    o_ref[...] = x.astype(jnp.float32).reshape(o_ref.shape)


def _scale_tc(x):
    rows = x.shape[0]
    blk = 512
    return pl.pallas_call(
        _scale_body,
        grid=(rows // blk,),
        in_specs=[pl.BlockSpec((blk, _SL, 128), lambda i: (i, 0, 0))],
        out_specs=pl.BlockSpec((blk // 8, 8, _D), lambda i: (i, 0, 0)),
        out_shape=jax.ShapeDtypeStruct((rows // 8, 8, _D), jnp.float32),
    )(x)


def kernel(input_ids, embed_table):
    idx = input_ids.reshape(_NW, _BPW)
    table = embed_table.reshape(_VOCAB, _SL, 128)
    rows = _embed_sc(idx, table)
    out = _scale_tc(rows)
    batch, seq = input_ids.shape
    return out.reshape(batch, seq, _D)
